# baseline (speedup 1.0000x reference)
import functools

import jax
import jax.numpy as jnp
from jax import lax
from jax.experimental import pallas as pl
from jax.experimental.pallas import tpu as pltpu
from jax.experimental.pallas import tpu_sc as plsc

D = 8192
B = 4 * 2048
NC, NS = 2, 16
NW = NC * NS
B_PER_W = B // NW
C = 6
NFULL = B_PER_W // C
REM = B_PER_W - NFULL * C
NCHUNK = NFULL + 1

_mesh = plsc.VectorSubcoreMesh(core_axis_name="c", subcore_axis_name="s")


@functools.partial(
    pl.kernel,
    mesh=_mesh,
    out_type=jax.ShapeDtypeStruct((NW, B_PER_W, D), jnp.float32),
    compiler_params=pltpu.CompilerParams(use_tc_tiling_on_sc=False),
    scratch_types=[
        pltpu.VMEM((NCHUNK, C), jnp.int32),
        pltpu.VMEM((C, D), jnp.float32),
        pltpu.VMEM((C, D), jnp.float32),
        pltpu.SemaphoreType.DMA,
        pltpu.SemaphoreType.DMA,
        pltpu.SemaphoreType.DMA,
        pltpu.SemaphoreType.DMA,
    ],
)
def _gather_sc(x_hbm, table_hbm, out_hbm, idx_v, buf0, buf1, g0, g1, s0, s1):
    wid = lax.axis_index("s") * NC + lax.axis_index("c")
    pltpu.sync_copy(x_hbm.at[wid], idx_v)

    bufs = (buf0, buf1)
    gsems = (g0, g1)
    ssems = (s0, s1)

    def g_start(c, b):
        pltpu.async_copy(table_hbm.at[idx_v.at[c]], bufs[b], gsems[b])

    def g_wait(b):
        pltpu.make_async_copy(table_hbm.at[idx_v.at[0]], bufs[b], gsems[b]).wait()

    def w_start(c, b):
        pltpu.async_copy(bufs[b], out_hbm.at[wid, pl.ds(c * C, C)], ssems[b])

    def w_wait(b):
        pltpu.make_async_copy(bufs[b], out_hbm.at[wid, pl.ds(0, C)], ssems[b]).wait()

    g_start(0, 0)
    g_wait(0)
    w_start(0, 0)
    g_start(1, 1)

    def round_body(r, carry):
        c1 = 2 * r + 1
        g_wait(1)
        w_start(c1, 1)
        w_wait(0)
        g_start(c1 + 1, 0)
        g_wait(0)
        w_start(c1 + 1, 0)
        w_wait(1)
        g_start(c1 + 2, 1)
        return carry

    lax.fori_loop(0, (NFULL - 2) // 2, round_body, 0)

    g_wait(1)
    w_start(NFULL - 1, 1)
    w_wait(0)
    g_start(NFULL, 0)
    g_wait(0)
    pltpu.async_copy(
        buf0.at[pl.ds(0, REM)], out_hbm.at[wid, pl.ds(NFULL * C, REM)], s0
    )
    w_wait(1)
    pltpu.make_async_copy(
        buf0.at[pl.ds(0, REM)], out_hbm.at[wid, pl.ds(NFULL * C, REM)], s0
    ).wait()


def kernel(x, table):
    xf = x.reshape(NW, B_PER_W)
    xp = jnp.pad(xf, ((0, 0), (0, NCHUNK * C - B_PER_W)), mode="edge")
    out = _gather_sc(xp.reshape(NW, NCHUNK, C), table)
    return out.reshape(4, 2048, D)


# speedup vs baseline: 1.0025x; 1.0025x over previous
import functools

import jax
import jax.numpy as jnp
from jax import lax
from jax.experimental import pallas as pl
from jax.experimental.pallas import tpu as pltpu
from jax.experimental.pallas import tpu_sc as plsc

D = 8192
V = 8192
B = 4 * 2048
NC, NS = 2, 16
NW = NC * NS
B_PER_W = B // NW
C = 6
NFULL = B_PER_W // C
REM = B_PER_W - NFULL * C
NCHUNK = NFULL + 1

_mesh = plsc.VectorSubcoreMesh(core_axis_name="c", subcore_axis_name="s")


@functools.partial(
    pl.kernel,
    mesh=_mesh,
    out_type=jax.ShapeDtypeStruct((B, 8, 8, 128), jnp.float32),
    scratch_types=[
        pltpu.VMEM((NCHUNK, C), jnp.int32),
        pltpu.VMEM((C, 8, 8, 128), jnp.float32),
        pltpu.VMEM((C, 8, 8, 128), jnp.float32),
        pltpu.SemaphoreType.DMA,
        pltpu.SemaphoreType.DMA,
        pltpu.SemaphoreType.DMA,
        pltpu.SemaphoreType.DMA,
    ],
)
def _gather_sc(x_hbm, table_hbm, out_hbm, idx_v, buf0, buf1, g0, g1, s0, s1):
    wid = lax.axis_index("s") * NC + lax.axis_index("c")
    base = wid * B_PER_W
    pltpu.sync_copy(x_hbm.at[wid], idx_v)

    bufs = (buf0, buf1)
    gsems = (g0, g1)
    ssems = (s0, s1)

    def g_start(c, b):
        pltpu.async_copy(table_hbm.at[idx_v.at[c]], bufs[b], gsems[b])

    def g_wait(b):
        pltpu.make_async_copy(table_hbm.at[idx_v.at[0]], bufs[b], gsems[b]).wait()

    def w_start(c, b):
        pltpu.async_copy(bufs[b], out_hbm.at[pl.ds(base + c * C, C)], ssems[b])

    def w_wait(b):
        pltpu.make_async_copy(bufs[b], out_hbm.at[pl.ds(base, C)], ssems[b]).wait()

    g_start(0, 0)
    g_wait(0)
    w_start(0, 0)
    g_start(1, 1)

    def round_body(r, carry):
        c1 = 2 * r + 1
        g_wait(1)
        w_start(c1, 1)
        w_wait(0)
        g_start(c1 + 1, 0)
        g_wait(0)
        w_start(c1 + 1, 0)
        w_wait(1)
        g_start(c1 + 2, 1)
        return carry

    lax.fori_loop(0, (NFULL - 2) // 2, round_body, 0)

    g_wait(1)
    w_start(NFULL - 1, 1)
    w_wait(0)
    g_start(NFULL, 0)
    g_wait(0)
    pltpu.async_copy(
        buf0.at[pl.ds(0, REM)],
        out_hbm.at[pl.ds(base + NFULL * C, REM)],
        s0,
    )
    w_wait(1)
    pltpu.make_async_copy(
        buf0.at[pl.ds(0, REM)],
        out_hbm.at[pl.ds(base + NFULL * C, REM)],
        s0,
    ).wait()


def kernel(x, table):
    xf = x.reshape(NW, B_PER_W)
    xp = jnp.pad(xf, ((0, 0), (0, NCHUNK * C - B_PER_W)), mode="edge")
    t4 = table.reshape(V, 8, 8, 128)
    out = _gather_sc(xp.reshape(NW, NCHUNK, C), t4)
    return out.reshape(4, 2048, D)


# speedup vs baseline: 3.0140x; 3.0064x over previous
import functools

import jax
import jax.numpy as jnp
from jax import lax
from jax.experimental import pallas as pl
from jax.experimental.pallas import tpu as pltpu
from jax.experimental.pallas import tpu_sc as plsc

D = 8192
B = 4 * 2048
NC, NS = 2, 16
NW = NC * NS
B_PER_W = B // NW
C = 8
NCHUNK = B_PER_W // C

_mesh = plsc.VectorSubcoreMesh(core_axis_name="c", subcore_axis_name="s")


@functools.partial(
    pl.kernel,
    mesh=_mesh,
    out_type=jax.ShapeDtypeStruct((NW, NCHUNK, C, D), jnp.float32),
    scratch_types=[
        pltpu.VMEM((NCHUNK, C), jnp.int32),
        pltpu.VMEM((C, D), jnp.float32),
        pltpu.SemaphoreType.DMA,
    ],
)
def _gather_sc(x_hbm, table_hbm, out_hbm, idx_v, rows_v, gsem):
    wid = lax.axis_index("s") * NC + lax.axis_index("c")
    pltpu.sync_copy(x_hbm.at[wid], idx_v)

    def step(c, carry):
        pltpu.async_copy(table_hbm.at[idx_v.at[c]], rows_v, gsem).wait()
        pltpu.sync_copy(rows_v, out_hbm.at[wid, c])
        return carry

    lax.fori_loop(0, NCHUNK, step, 0)


def kernel(x, table):
    xf = x.reshape(NW, NCHUNK, C)
    out = _gather_sc(xf, table)
    return out.reshape(4, 2048, D)
